# baseline (speedup 1.0000x reference)
import jax
import jax.numpy as jnp
from jax.experimental import pallas as pl

_DIM = 1024
_SCALE = _DIM ** (-0.5)
_BLOCK_ROWS = 512


def _scale_kernel(emb_ref, out_ref):
    out_ref[...] = emb_ref[...] * _SCALE


def kernel(x, emb):
    seq_len = x.shape[1]
    rows = emb.shape[0]
    assert seq_len == rows
    grid = rows // _BLOCK_ROWS
    return pl.pallas_call(
        _scale_kernel,
        grid=(grid,),
        in_specs=[pl.BlockSpec((_BLOCK_ROWS, _DIM), lambda i: (i, 0))],
        out_specs=pl.BlockSpec((_BLOCK_ROWS, _DIM), lambda i: (i, 0)),
        out_shape=jax.ShapeDtypeStruct((rows, _DIM), emb.dtype),
    )(emb)


# speedup vs baseline: 1.1811x; 1.1811x over previous
import jax
import jax.numpy as jnp
from jax.experimental import pallas as pl

_DIM = 1024
_SCALE = _DIM ** (-0.5)
_BLOCK_ROWS = 2048


def _scale_kernel(emb_ref, out_ref):
    out_ref[...] = emb_ref[...] * _SCALE


def kernel(x, emb):
    seq_len = x.shape[1]
    rows = emb.shape[0]
    assert seq_len == rows
    grid = rows // _BLOCK_ROWS
    return pl.pallas_call(
        _scale_kernel,
        grid=(grid,),
        in_specs=[pl.BlockSpec((_BLOCK_ROWS, _DIM), lambda i: (i, 0))],
        out_specs=pl.BlockSpec((_BLOCK_ROWS, _DIM), lambda i: (i, 0)),
        out_shape=jax.ShapeDtypeStruct((rows, _DIM), emb.dtype),
    )(emb)
